# baseline (speedup 1.0000x reference)
import functools

import jax
import jax.numpy as jnp
from jax import lax
from jax.experimental import pallas as pl
from jax.experimental.pallas import tpu as pltpu
from jax.experimental.pallas import tpu_sc as plsc

N = 32768
D = 512
B = 16
L = 16
CH = 512
G = CH // 8
NBUF = 4


def _body(x_hbm, cum_hbm, out_hbm, cum_v, buf, res, sem):
    s = lax.axis_index("s")
    c = lax.axis_index("c")
    tc = c * 2 + lax.div(s, 8)
    j16 = lax.rem(s, 8) * L

    pltpu.sync_copy(cum_hbm, cum_v)
    cumv = cum_v[...]
    scs = [cumv[j] for j in range(B)]
    total = scs[B - 1]
    nch = (total + CH - 1) // CH

    zero = jnp.zeros((L,), jnp.float32)
    for b in range(B):
        res[b, :] = zero

    def dcopy(ci, slot):
        return pltpu.make_async_copy(
            x_hbm.at[pl.ds(ci * G, G), pl.ds(tc, 1), :, pl.ds(j16, L)],
            buf.at[pl.ds(slot * G, G)],
            sem.at[slot],
        )

    for k in range(NBUF):
        @pl.when(k < nch)
        def _():
            dcopy(jnp.int32(k), jnp.int32(k)).start()

    def chunk_step(ci, carry):
        slot = lax.rem(ci, NBUF)
        dcopy(ci, slot).wait()
        g0 = ci * CH
        g1 = jnp.minimum(g0 + CH, total)
        bg = slot * G - ci * G

        def accum(rl, rh):
            mid0 = jnp.minimum((rl + 7) & ~7, rh)
            mid1 = jnp.maximum(rh & ~7, mid0)

            def head(r, t):
                return t + buf[bg + (r >> 3), 0, r & 7]

            t0 = lax.fori_loop(rl, mid0, head, zero)

            gA = mid0 >> 3
            gB = mid1 >> 3
            pairs = (gB - gA) >> 1

            def blk2(kk, a):
                g = bg + gA + kk * 2
                return tuple(a[k] + (buf[g, 0, k] + buf[g + 1, 0, k])
                             for k in range(8))

            a = lax.fori_loop(0, pairs, blk2, (t0,) + (zero,) * 7)

            def blk1(gg, a):
                g = bg + gg
                return tuple(a[k] + buf[g, 0, k] for k in range(8))

            a = lax.fori_loop(gA + pairs * 2, gB, blk1, a)

            def tail(r, t):
                return t + buf[bg + (r >> 3), 0, r & 7]

            s4 = ((a[0] + a[1]) + (a[2] + a[3])) + \
                 ((a[4] + a[5]) + (a[6] + a[7]))
            return lax.fori_loop(mid1, rh, tail, s4)

        seg0 = jnp.int32(0)
        e = total
        for j in range(B):
            seg0 = seg0 + (scs[j] <= g0).astype(jnp.int32)
        for j in range(B - 1, -1, -1):
            e = jnp.where(scs[j] > g0, scs[j], e)

        @pl.when(e >= g1)
        def _():
            res[seg0] = res[seg0] + accum(g0, g1)

        @pl.when(e < g1)
        def _():
            for b in range(B):
                s_b = scs[b - 1] if b else jnp.int32(0)
                e_b = scs[b]
                lo = jnp.minimum(jnp.maximum(s_b, g0), g1)
                hi = jnp.minimum(jnp.maximum(e_b, g0), g1)
                res[b, :] = res[b, :] + accum(lo, hi)

        @pl.when(ci + NBUF < nch)
        def _():
            dcopy(ci + NBUF, slot).start()

        return carry

    lax.fori_loop(0, nch, chunk_step, jnp.int32(0))

    for b in range(B):
        n_b = scs[b] - (scs[b - 1] if b else jnp.int32(0))
        res[b, :] = res[b, :] / n_b.astype(jnp.float32)

    pltpu.sync_copy(res, out_hbm.at[:, pl.ds(tc * 128 + j16, L)])


def kernel(x, lengths):
    f = functools.partial(
        pl.kernel,
        mesh=plsc.VectorSubcoreMesh(core_axis_name="c", subcore_axis_name="s"),
        out_type=jax.ShapeDtypeStruct((B, D), jnp.float32),
        scratch_types=[
            pltpu.VMEM((B,), jnp.int32),
            pltpu.VMEM((NBUF * G, 1, 8, L), jnp.float32),
            pltpu.VMEM((B, L), jnp.float32),
            pltpu.SemaphoreType.DMA((NBUF,)),
        ],
        compiler_params=pltpu.CompilerParams(use_tc_tiling_on_sc=False),
    )(_body)
    x4 = jnp.transpose(jnp.reshape(x, (N // 8, 8, 4, 128)), (0, 2, 1, 3))
    cum = jnp.cumsum(lengths)
    return f(x4, cum)


# speedup vs baseline: 2.0774x; 2.0774x over previous
import functools

import jax
import jax.numpy as jnp
from jax import lax
from jax.experimental import pallas as pl
from jax.experimental.pallas import tpu as pltpu
from jax.experimental.pallas import tpu_sc as plsc

N = 32768
D = 512
B = 16
L = 16
CH = 512
G = CH // 8
NBUF = 4


def _body(x_hbm, cum_hbm, out_hbm, cum_v, buf, res, sem):
    s = lax.axis_index("s")
    c = lax.axis_index("c")
    tc = c * 2 + lax.div(s, 8)
    j16 = lax.rem(s, 8) * L

    pltpu.sync_copy(cum_hbm, cum_v)
    cumv = cum_v[...]
    scs = [cumv[j] for j in range(B)]
    total = scs[B - 1]
    nch = (total + CH - 1) // CH

    zero = jnp.zeros((L,), jnp.float32)
    for b in range(B):
        res[b, :] = zero

    def dcopy(ci, slot):
        return pltpu.make_async_copy(
            x_hbm.at[pl.ds(ci * G, G), pl.ds(tc, 1), :, pl.ds(j16, L)],
            buf.at[pl.ds(slot * G, G)],
            sem.at[slot],
        )

    for k in range(NBUF):
        @pl.when(k < nch)
        def _():
            dcopy(jnp.int32(k), jnp.int32(k)).start()

    def chunk_step(ci, carry):
        slot = lax.rem(ci, NBUF)
        dcopy(ci, slot).wait()
        g0 = ci * CH
        g1 = jnp.minimum(g0 + CH, total)
        bg = slot * G - ci * G

        def accum(rl, rh):
            mid0 = jnp.minimum((rl + 7) & ~7, rh)
            mid1 = jnp.maximum(rh & ~7, mid0)

            def head(r, t):
                return t + buf[bg + (r >> 3), 0, r & 7]

            t0 = lax.fori_loop(rl, mid0, head, zero)

            def blk(gg, a):
                g = bg + gg
                return tuple(a[k] + buf[g, 0, k] for k in range(8))

            a = lax.fori_loop(mid0 >> 3, mid1 >> 3, blk, (t0,) + (zero,) * 7)

            def tail(r, t):
                return t + buf[bg + (r >> 3), 0, r & 7]

            s4 = ((a[0] + a[1]) + (a[2] + a[3])) + \
                 ((a[4] + a[5]) + (a[6] + a[7]))
            return lax.fori_loop(mid1, rh, tail, s4)

        seg0 = jnp.int32(0)
        segN = jnp.int32(0)
        for j in range(B):
            seg0 = seg0 + (scs[j] <= g0).astype(jnp.int32)
            segN = segN + (scs[j] < g1).astype(jnp.int32)

        def seg_body(b, carry2):
            s_b = jnp.int32(0)
            e_b = scs[0]
            for j in range(1, B):
                isj = b == j
                s_b = jnp.where(isj, scs[j - 1], s_b)
                e_b = jnp.where(isj, scs[j], e_b)
            lo = jnp.minimum(jnp.maximum(s_b, g0), g1)
            hi = jnp.minimum(jnp.maximum(e_b, g0), g1)
            res[b] = res[b] + accum(lo, hi)
            return carry2

        lax.fori_loop(seg0, segN + 1, seg_body, jnp.int32(0))

        @pl.when(ci + NBUF < nch)
        def _():
            dcopy(ci + NBUF, slot).start()

        return carry

    lax.fori_loop(0, nch, chunk_step, jnp.int32(0))

    for b in range(B):
        n_b = scs[b] - (scs[b - 1] if b else jnp.int32(0))
        res[b, :] = res[b, :] / n_b.astype(jnp.float32)

    pltpu.sync_copy(res, out_hbm.at[:, pl.ds(tc * 128 + j16, L)])


def kernel(x, lengths):
    f = functools.partial(
        pl.kernel,
        mesh=plsc.VectorSubcoreMesh(core_axis_name="c", subcore_axis_name="s"),
        out_type=jax.ShapeDtypeStruct((B, D), jnp.float32),
        scratch_types=[
            pltpu.VMEM((B,), jnp.int32),
            pltpu.VMEM((NBUF * G, 1, 8, L), jnp.float32),
            pltpu.VMEM((B, L), jnp.float32),
            pltpu.SemaphoreType.DMA((NBUF,)),
        ],
        compiler_params=pltpu.CompilerParams(use_tc_tiling_on_sc=False),
    )(_body)
    x4 = jnp.transpose(jnp.reshape(x, (N // 8, 8, 4, 128)), (0, 2, 1, 3))
    cum = jnp.cumsum(lengths)
    return f(x4, cum)
